# baseline (speedup 1.0000x reference)
import functools

import jax
import jax.numpy as jnp
from jax import lax
from jax.experimental import pallas as pl
from jax.experimental.pallas import tpu as pltpu
from jax.experimental.pallas import tpu_sc as plsc

N = 10000
E = 320000
F = 128
G = 64
C = 10

NC = 2
NS = 16
NW = NC * NS
EW = E // NW
K = 125
CH = EW // K
IB = 16
NPAD = 10240
RPT = NPAD // NS
ZR = 32

BLK = 1000
GRID = N // BLK


@functools.cache
def _sc_build():
    mesh = plsc.VectorSubcoreMesh(core_axis_name="c", subcore_axis_name="s")

    @functools.partial(
        pl.kernel,
        out_type=jax.ShapeDtypeStruct((NC, NPAD, F), jnp.float32),
        mesh=mesh,
        scratch_types=[
            pltpu.VMEM_SHARED((NPAD, F), jnp.float32),
            pltpu.VMEM((CH, K), jnp.int32),
            pltpu.VMEM((K, F), jnp.float32),
            pltpu.VMEM((ZR, F), jnp.float32),
        ],
    )
    def sc_degree(dst_hbm, out_hbm, acc, idx, buf, zbuf):
        c = lax.axis_index("c")
        s = lax.axis_index("s")
        wid = c * NS + s

        zv = jnp.zeros((16,), jnp.float32)

        def zbody(i, carry):
            for q in range(F // 16):
                zbuf[i, pl.ds(q * 16, 16)] = zv
            return carry

        lax.fori_loop(0, ZR, zbody, 0)
        for t in range(RPT // ZR):
            pltpu.sync_copy(zbuf, acc.at[pl.ds(s * RPT + t * ZR, ZR)])
        pltpu.sync_copy(dst_hbm.at[wid], idx)

        ov = jnp.ones((16,), jnp.float32)

        def obody(i, carry):
            for q in range(F // 16):
                buf[i, pl.ds(q * 16, 16)] = ov
            return carry

        lax.fori_loop(0, K, obody, 0)
        plsc.subcore_barrier()

        def chunk(j, carry):
            pltpu.sync_copy(buf, acc.at[idx.at[j]], add=True)
            return carry

        lax.fori_loop(0, CH, chunk, 0)
        plsc.subcore_barrier()
        pltpu.sync_copy(acc.at[pl.ds(s * RPT, RPT)],
                        out_hbm.at[c, pl.ds(s * RPT, RPT)])

    @functools.partial(
        pl.kernel,
        out_type=jax.ShapeDtypeStruct((NC, NPAD, F), jnp.float32),
        mesh=mesh,
        scratch_types=[
            pltpu.VMEM_SHARED((NPAD, F), jnp.float32),
            pltpu.VMEM((IB, K), jnp.int32),
            pltpu.VMEM((IB, K), jnp.int32),
            pltpu.VMEM((K, F), jnp.float32),
            pltpu.VMEM((K, F), jnp.float32),
            pltpu.VMEM((ZR, F), jnp.float32),
            pltpu.SemaphoreType.DMA,
            pltpu.SemaphoreType.DMA,
            pltpu.SemaphoreType.DMA,
            pltpu.SemaphoreType.DMA,
        ],
    )
    def sc_scatter(hs_hbm, src_hbm, dst_hbm, out_hbm,
                   acc, sidx, didx, rows0, rows1, zbuf,
                   sem0, sem1, ssem0, ssem1):
        c = lax.axis_index("c")
        s = lax.axis_index("s")
        wid = c * NS + s

        zv = jnp.zeros((16,), jnp.float32)

        def zbody(i, carry):
            for q in range(F // 16):
                zbuf[i, pl.ds(q * 16, 16)] = zv
            return carry

        lax.fori_loop(0, ZR, zbody, 0)
        for t in range(RPT // ZR):
            pltpu.sync_copy(zbuf, acc.at[pl.ds(s * RPT + t * ZR, ZR)])
        plsc.subcore_barrier()

        def stage(t, carry):
            pltpu.sync_copy(src_hbm.at[wid, pl.ds(t * IB, IB)], sidx)
            pltpu.sync_copy(dst_hbm.at[wid, pl.ds(t * IB, IB)], didx)
            bufs = (rows0, rows1)
            gsems = (sem0, sem1)
            ssems = (ssem0, ssem1)
            pend_g = pltpu.async_copy(hs_hbm.at[sidx.at[0]], rows0, sem0)
            pend_s = [None, None]
            for b in range(IB):
                p = b % 2
                pend_g.wait()
                pend_s[p] = pltpu.async_copy(
                    bufs[p], acc.at[didx.at[b]], ssems[p], add=True)
                if b + 1 < IB:
                    q = (b + 1) % 2
                    if pend_s[q] is not None:
                        pend_s[q].wait()
                        pend_s[q] = None
                    pend_g = pltpu.async_copy(
                        hs_hbm.at[sidx.at[b + 1]], bufs[q], gsems[q])
            for p in range(2):
                if pend_s[p] is not None:
                    pend_s[p].wait()
            return carry

        lax.fori_loop(0, CH // IB, stage, 0)
        plsc.subcore_barrier()
        pltpu.sync_copy(acc.at[pl.ds(s * RPT, RPT)],
                        out_hbm.at[c, pl.ds(s * RPT, RPT)])

    return sc_degree, sc_scatter


def _sc_degree(dst_r):
    return _sc_build()[0](dst_r)


def _sc_scatter(hs, src_r, dst_r):
    return _sc_build()[1](hs, src_r, dst_r)


def _tc_prep(deg_parts, x, W, R):
    def body(dp_ref, x_ref, w_ref, r_ref, inv_ref, hr_ref, hs_ref):
        d = dp_ref[0, :, 0:1] + dp_ref[1, :, 0:1] + 1.0
        iv = lax.rsqrt(d)
        inv_ref[...] = iv
        xb = x_ref[...]
        h = jnp.dot(xb, w_ref[...], preferred_element_type=jnp.float32)
        hr_ref[...] = jnp.dot(xb, r_ref[...], preferred_element_type=jnp.float32)
        hs_ref[...] = h * iv

    return pl.pallas_call(
        body,
        grid=(GRID,),
        in_specs=[
            pl.BlockSpec((NC, BLK, F), lambda i: (0, i, 0)),
            pl.BlockSpec((BLK, F), lambda i: (i, 0)),
            pl.BlockSpec((F, F), lambda i: (0, 0)),
            pl.BlockSpec((F, F), lambda i: (0, 0)),
        ],
        out_specs=[
            pl.BlockSpec((BLK, 1), lambda i: (i, 0)),
            pl.BlockSpec((BLK, F), lambda i: (i, 0)),
            pl.BlockSpec((BLK, F), lambda i: (i, 0)),
        ],
        out_shape=[
            jax.ShapeDtypeStruct((N, 1), jnp.float32),
            jax.ShapeDtypeStruct((N, F), jnp.float32),
            jax.ShapeDtypeStruct((N, F), jnp.float32),
        ],
    )(deg_parts, x, W, R)


def _tc_combine(S, hs, r, inv, b, Wn, Rn):
    def body(s_ref, hs_ref_in, r_ref, inv_ref, b_ref, w_ref, rr_ref,
             rn_ref, hs_ref):
        iv = inv_ref[...]
        stot = s_ref[0] + s_ref[1]
        hout = jnp.maximum(
            (stot + hs_ref_in[...]) * iv + r_ref[...] + b_ref[...], 0.0)
        hn = jnp.dot(hout, w_ref[...], preferred_element_type=jnp.float32)
        rn_ref[...] = jnp.dot(hout, rr_ref[...],
                              preferred_element_type=jnp.float32)
        hs_ref[...] = hn * iv

    return pl.pallas_call(
        body,
        grid=(GRID,),
        in_specs=[
            pl.BlockSpec((NC, BLK, F), lambda i: (0, i, 0)),
            pl.BlockSpec((BLK, F), lambda i: (i, 0)),
            pl.BlockSpec((BLK, F), lambda i: (i, 0)),
            pl.BlockSpec((BLK, 1), lambda i: (i, 0)),
            pl.BlockSpec((1, F), lambda i: (0, 0)),
            pl.BlockSpec((F, F), lambda i: (0, 0)),
            pl.BlockSpec((F, F), lambda i: (0, 0)),
        ],
        out_specs=[pl.BlockSpec((BLK, F), lambda i: (i, 0))] * 2,
        out_shape=[jax.ShapeDtypeStruct((N, F), jnp.float32)] * 2,
    )(S, hs, r, inv, b, Wn, Rn)


def _tc_final(S, hs, r, inv, b, batch2d, W1, b1, W2, b2):
    def body(s_ref, hs_ref_in, r_ref, inv_ref, b_ref, bt_ref,
             w1_ref, b1_ref, w2_ref, b2_ref,
             logp_ref, last_ref, pacc, cacc):
        i = pl.program_id(0)
        iv = inv_ref[...]
        stot = s_ref[0] + s_ref[1]
        hout = jnp.maximum(
            (stot + hs_ref_in[...]) * iv + r_ref[...] + b_ref[...], 0.0)
        last_ref[...] = hout

        gi = lax.broadcasted_iota(jnp.int32, (BLK, G), 1)
        oh = (bt_ref[...] == gi).astype(jnp.float32)
        dn = (((0,), (0,)), ((), ()))
        psum = lax.dot_general(oh, hout, dn,
                               preferred_element_type=jnp.float32)
        csum = lax.dot_general(oh, jnp.ones((BLK, F), jnp.float32), dn,
                               preferred_element_type=jnp.float32)

        @pl.when(i == 0)
        def _():
            pacc[...] = psum
            cacc[...] = csum

        @pl.when(i > 0)
        def _():
            pacc[...] += psum
            cacc[...] += csum

        @pl.when(i == GRID - 1)
        def _():
            pooled = pacc[...] / jnp.maximum(cacc[...], 1.0)
            o1 = jnp.maximum(
                jnp.dot(pooled, w1_ref[...],
                        preferred_element_type=jnp.float32) + b1_ref[...], 0.0)
            logits = jnp.dot(o1, w2_ref[...],
                             preferred_element_type=jnp.float32) + b2_ref[...]
            m = jnp.max(logits, axis=-1, keepdims=True)
            sh = logits - m
            lse = jnp.log(jnp.sum(jnp.exp(sh), axis=-1, keepdims=True))
            logp_ref[...] = sh - lse

    return pl.pallas_call(
        body,
        grid=(GRID,),
        in_specs=[
            pl.BlockSpec((NC, BLK, F), lambda i: (0, i, 0)),
            pl.BlockSpec((BLK, F), lambda i: (i, 0)),
            pl.BlockSpec((BLK, F), lambda i: (i, 0)),
            pl.BlockSpec((BLK, 1), lambda i: (i, 0)),
            pl.BlockSpec((1, F), lambda i: (0, 0)),
            pl.BlockSpec((BLK, 1), lambda i: (i, 0)),
            pl.BlockSpec((F, F), lambda i: (0, 0)),
            pl.BlockSpec((1, F), lambda i: (0, 0)),
            pl.BlockSpec((F, C), lambda i: (0, 0)),
            pl.BlockSpec((1, C), lambda i: (0, 0)),
        ],
        out_specs=[
            pl.BlockSpec((G, C), lambda i: (0, 0)),
            pl.BlockSpec((BLK, F), lambda i: (i, 0)),
        ],
        out_shape=[
            jax.ShapeDtypeStruct((G, C), jnp.float32),
            jax.ShapeDtypeStruct((N, F), jnp.float32),
        ],
        scratch_shapes=[
            pltpu.VMEM((G, F), jnp.float32),
            pltpu.VMEM((G, F), jnp.float32),
        ],
    )(S, hs, r, inv, b, batch2d, W1, b1, W2, b2)


def kernel(x, edge_index, batch, W0, R0, b0, W1, R1, b1, W2, R2, b2,
           lin1_W, lin1_b, lin2_W, lin2_b):
    src_r = edge_index[0].reshape(NW, CH, K)
    dst_r = edge_index[1].reshape(NW, CH, K)

    deg_parts = _sc_degree(dst_r)
    inv, r0, hs0 = _tc_prep(deg_parts, x, W0, R0)

    S0 = _sc_scatter(hs0, src_r, dst_r)
    r1, hs1 = _tc_combine(S0, hs0, r0, inv, b0.reshape(1, F), W1, R1)
    S1 = _sc_scatter(hs1, src_r, dst_r)
    r2, hs2 = _tc_combine(S1, hs1, r1, inv, b1.reshape(1, F), W2, R2)
    S2 = _sc_scatter(hs2, src_r, dst_r)

    logp, last = _tc_final(
        S2, hs2, r2, inv, b2.reshape(1, F),
        batch.astype(jnp.int32).reshape(N, 1),
        lin1_W, lin1_b.reshape(1, F), lin2_W, lin2_b.reshape(1, C))
    return (logp, last)


# speedup vs baseline: 1.0080x; 1.0080x over previous
import functools

import jax
import jax.numpy as jnp
from jax import lax
from jax.experimental import pallas as pl
from jax.experimental.pallas import tpu as pltpu
from jax.experimental.pallas import tpu_sc as plsc

N = 10000
E = 320000
F = 128
G = 64
C = 10

NC = 2
NS = 16
NW = NC * NS
EW = E // NW
K = 125
CH = EW // K
IB = 16
NPAD = 10240
RPT = NPAD // NS
ZR = 64

BLK = 1000
GRID = N // BLK


@functools.cache
def _sc_build():
    mesh = plsc.VectorSubcoreMesh(core_axis_name="c", subcore_axis_name="s")

    @functools.partial(
        pl.kernel,
        out_type=jax.ShapeDtypeStruct((NC, NPAD, F), jnp.float32),
        mesh=mesh,
        scratch_types=[
            pltpu.VMEM_SHARED((NPAD, F), jnp.float32),
            pltpu.VMEM((CH, K), jnp.int32),
            pltpu.VMEM((K, F), jnp.float32),
            pltpu.VMEM((ZR, F), jnp.float32),
        ],
    )
    def sc_degree(dst_hbm, out_hbm, acc, idx, buf, zbuf):
        c = lax.axis_index("c")
        s = lax.axis_index("s")
        wid = c * NS + s

        zv = jnp.zeros((16,), jnp.float32)

        def zbody(i, carry):
            for q in range(F // 16):
                zbuf[i, pl.ds(q * 16, 16)] = zv
            return carry

        lax.fori_loop(0, ZR, zbody, 0)
        for t in range(RPT // ZR):
            pltpu.sync_copy(zbuf, acc.at[pl.ds(s * RPT + t * ZR, ZR)])
        pltpu.sync_copy(dst_hbm.at[wid], idx)

        ov = jnp.ones((16,), jnp.float32)

        def obody(i, carry):
            for q in range(F // 16):
                buf[i, pl.ds(q * 16, 16)] = ov
            return carry

        lax.fori_loop(0, K, obody, 0)
        plsc.subcore_barrier()

        def chunk(j, carry):
            pltpu.sync_copy(buf, acc.at[idx.at[j]], add=True)
            return carry

        lax.fori_loop(0, CH, chunk, 0)
        plsc.subcore_barrier()
        pltpu.sync_copy(acc.at[pl.ds(s * RPT, RPT)],
                        out_hbm.at[c, pl.ds(s * RPT, RPT)])

    @functools.partial(
        pl.kernel,
        out_type=jax.ShapeDtypeStruct((NC, NPAD, F), jnp.float32),
        mesh=mesh,
        scratch_types=[
            pltpu.VMEM_SHARED((NPAD, F), jnp.float32),
            pltpu.VMEM((IB, K), jnp.int32),
            pltpu.VMEM((IB, K), jnp.int32),
            pltpu.VMEM((K, F), jnp.float32),
            pltpu.VMEM((K, F), jnp.float32),
            pltpu.VMEM((ZR, F), jnp.float32),
            pltpu.SemaphoreType.DMA,
            pltpu.SemaphoreType.DMA,
            pltpu.SemaphoreType.DMA,
            pltpu.SemaphoreType.DMA,
        ],
    )
    def sc_scatter(hs_hbm, src_hbm, dst_hbm, out_hbm,
                   acc, sidx, didx, rows0, rows1, zbuf,
                   sem0, sem1, ssem0, ssem1):
        c = lax.axis_index("c")
        s = lax.axis_index("s")
        wid = c * NS + s

        zv = jnp.zeros((16,), jnp.float32)

        def zbody(i, carry):
            for q in range(F // 16):
                zbuf[i, pl.ds(q * 16, 16)] = zv
            return carry

        lax.fori_loop(0, ZR, zbody, 0)
        for t in range(RPT // ZR):
            pltpu.sync_copy(zbuf, acc.at[pl.ds(s * RPT + t * ZR, ZR)])
        plsc.subcore_barrier()

        def stage(t, carry):
            pltpu.sync_copy(src_hbm.at[wid, pl.ds(t * IB, IB)], sidx)
            pltpu.sync_copy(dst_hbm.at[wid, pl.ds(t * IB, IB)], didx)
            bufs = (rows0, rows1)
            gsems = (sem0, sem1)
            pend_g = pltpu.async_copy(hs_hbm.at[sidx.at[0]], rows0, sem0)
            for b in range(IB):
                p = b % 2
                pend_g.wait()
                if b + 1 < IB:
                    q = (b + 1) % 2
                    pend_g = pltpu.async_copy(
                        hs_hbm.at[sidx.at[b + 1]], bufs[q], gsems[q])
                pltpu.sync_copy(bufs[p], acc.at[didx.at[b]], add=True)
            return carry

        lax.fori_loop(0, CH // IB, stage, 0)
        plsc.subcore_barrier()
        pltpu.sync_copy(acc.at[pl.ds(s * RPT, RPT)],
                        out_hbm.at[c, pl.ds(s * RPT, RPT)])

    return sc_degree, sc_scatter


def _sc_degree(dst_r):
    return _sc_build()[0](dst_r)


def _sc_scatter(hs, src_r, dst_r):
    return _sc_build()[1](hs, src_r, dst_r)


def _tc_prep(deg_parts, x, W, R):
    def body(dp_ref, x_ref, w_ref, r_ref, inv_ref, hr_ref, hs_ref):
        d = dp_ref[0, :, 0:1] + dp_ref[1, :, 0:1] + 1.0
        iv = lax.rsqrt(d)
        inv_ref[...] = iv
        xb = x_ref[...]
        h = jnp.dot(xb, w_ref[...], preferred_element_type=jnp.float32)
        hr_ref[...] = jnp.dot(xb, r_ref[...], preferred_element_type=jnp.float32)
        hs_ref[...] = h * iv

    return pl.pallas_call(
        body,
        grid=(GRID,),
        in_specs=[
            pl.BlockSpec((NC, BLK, F), lambda i: (0, i, 0)),
            pl.BlockSpec((BLK, F), lambda i: (i, 0)),
            pl.BlockSpec((F, F), lambda i: (0, 0)),
            pl.BlockSpec((F, F), lambda i: (0, 0)),
        ],
        out_specs=[
            pl.BlockSpec((BLK, 1), lambda i: (i, 0)),
            pl.BlockSpec((BLK, F), lambda i: (i, 0)),
            pl.BlockSpec((BLK, F), lambda i: (i, 0)),
        ],
        out_shape=[
            jax.ShapeDtypeStruct((N, 1), jnp.float32),
            jax.ShapeDtypeStruct((N, F), jnp.float32),
            jax.ShapeDtypeStruct((N, F), jnp.float32),
        ],
    )(deg_parts, x, W, R)


def _tc_combine(S, hs, r, inv, b, Wn, Rn):
    def body(s_ref, hs_ref_in, r_ref, inv_ref, b_ref, w_ref, rr_ref,
             rn_ref, hs_ref):
        iv = inv_ref[...]
        stot = s_ref[0] + s_ref[1]
        hout = jnp.maximum(
            (stot + hs_ref_in[...]) * iv + r_ref[...] + b_ref[...], 0.0)
        hn = jnp.dot(hout, w_ref[...], preferred_element_type=jnp.float32)
        rn_ref[...] = jnp.dot(hout, rr_ref[...],
                              preferred_element_type=jnp.float32)
        hs_ref[...] = hn * iv

    return pl.pallas_call(
        body,
        grid=(GRID,),
        in_specs=[
            pl.BlockSpec((NC, BLK, F), lambda i: (0, i, 0)),
            pl.BlockSpec((BLK, F), lambda i: (i, 0)),
            pl.BlockSpec((BLK, F), lambda i: (i, 0)),
            pl.BlockSpec((BLK, 1), lambda i: (i, 0)),
            pl.BlockSpec((1, F), lambda i: (0, 0)),
            pl.BlockSpec((F, F), lambda i: (0, 0)),
            pl.BlockSpec((F, F), lambda i: (0, 0)),
        ],
        out_specs=[pl.BlockSpec((BLK, F), lambda i: (i, 0))] * 2,
        out_shape=[jax.ShapeDtypeStruct((N, F), jnp.float32)] * 2,
    )(S, hs, r, inv, b, Wn, Rn)


def _tc_final(S, hs, r, inv, b, batch2d, W1, b1, W2, b2):
    def body(s_ref, hs_ref_in, r_ref, inv_ref, b_ref, bt_ref,
             w1_ref, b1_ref, w2_ref, b2_ref,
             logp_ref, last_ref, pacc, cacc):
        i = pl.program_id(0)
        iv = inv_ref[...]
        stot = s_ref[0] + s_ref[1]
        hout = jnp.maximum(
            (stot + hs_ref_in[...]) * iv + r_ref[...] + b_ref[...], 0.0)
        last_ref[...] = hout

        gi = lax.broadcasted_iota(jnp.int32, (BLK, G), 1)
        oh = (bt_ref[...] == gi).astype(jnp.float32)
        dn = (((0,), (0,)), ((), ()))
        psum = lax.dot_general(oh, hout, dn,
                               preferred_element_type=jnp.float32)
        csum = lax.dot_general(oh, jnp.ones((BLK, F), jnp.float32), dn,
                               preferred_element_type=jnp.float32)

        @pl.when(i == 0)
        def _():
            pacc[...] = psum
            cacc[...] = csum

        @pl.when(i > 0)
        def _():
            pacc[...] += psum
            cacc[...] += csum

        @pl.when(i == GRID - 1)
        def _():
            pooled = pacc[...] / jnp.maximum(cacc[...], 1.0)
            o1 = jnp.maximum(
                jnp.dot(pooled, w1_ref[...],
                        preferred_element_type=jnp.float32) + b1_ref[...], 0.0)
            logits = jnp.dot(o1, w2_ref[...],
                             preferred_element_type=jnp.float32) + b2_ref[...]
            m = jnp.max(logits, axis=-1, keepdims=True)
            sh = logits - m
            lse = jnp.log(jnp.sum(jnp.exp(sh), axis=-1, keepdims=True))
            logp_ref[...] = sh - lse

    return pl.pallas_call(
        body,
        grid=(GRID,),
        in_specs=[
            pl.BlockSpec((NC, BLK, F), lambda i: (0, i, 0)),
            pl.BlockSpec((BLK, F), lambda i: (i, 0)),
            pl.BlockSpec((BLK, F), lambda i: (i, 0)),
            pl.BlockSpec((BLK, 1), lambda i: (i, 0)),
            pl.BlockSpec((1, F), lambda i: (0, 0)),
            pl.BlockSpec((BLK, 1), lambda i: (i, 0)),
            pl.BlockSpec((F, F), lambda i: (0, 0)),
            pl.BlockSpec((1, F), lambda i: (0, 0)),
            pl.BlockSpec((F, C), lambda i: (0, 0)),
            pl.BlockSpec((1, C), lambda i: (0, 0)),
        ],
        out_specs=[
            pl.BlockSpec((G, C), lambda i: (0, 0)),
            pl.BlockSpec((BLK, F), lambda i: (i, 0)),
        ],
        out_shape=[
            jax.ShapeDtypeStruct((G, C), jnp.float32),
            jax.ShapeDtypeStruct((N, F), jnp.float32),
        ],
        scratch_shapes=[
            pltpu.VMEM((G, F), jnp.float32),
            pltpu.VMEM((G, F), jnp.float32),
        ],
    )(S, hs, r, inv, b, batch2d, W1, b1, W2, b2)


def kernel(x, edge_index, batch, W0, R0, b0, W1, R1, b1, W2, R2, b2,
           lin1_W, lin1_b, lin2_W, lin2_b):
    src_r = edge_index[0].reshape(NW, CH, K)
    dst_r = edge_index[1].reshape(NW, CH, K)

    deg_parts = _sc_degree(dst_r)
    inv, r0, hs0 = _tc_prep(deg_parts, x, W0, R0)

    S0 = _sc_scatter(hs0, src_r, dst_r)
    r1, hs1 = _tc_combine(S0, hs0, r0, inv, b0.reshape(1, F), W1, R1)
    S1 = _sc_scatter(hs1, src_r, dst_r)
    r2, hs2 = _tc_combine(S1, hs1, r1, inv, b1.reshape(1, F), W2, R2)
    S2 = _sc_scatter(hs2, src_r, dst_r)

    logp, last = _tc_final(
        S2, hs2, r2, inv, b2.reshape(1, F),
        batch.astype(jnp.int32).reshape(N, 1),
        lin1_W, lin1_b.reshape(1, F), lin2_W, lin2_b.reshape(1, C))
    return (logp, last)
